# baseline (speedup 1.0000x reference)
import functools
import math

import jax
import jax.numpy as jnp
from jax import lax
from jax.experimental import pallas as pl
from jax.experimental.pallas import tpu as pltpu
from jax.experimental.pallas import tpu_sc as plsc

_D = 128
_SCALE = math.sqrt(128.0)
_BATCH = 16384
_SEQ = 50
_NW = 32
_BPW = _BATCH // _NW
_C = 128
_KPS = _BPW // _C
_NCHUNK = _SEQ * _KPS
_NBUF = 3
_LANES = 16
_RU = 4


def _scale_chunk(rows_ref, buf):
    def row_body(r0, carry):
        for u in range(_RU):
            for j in range(_D // _LANES):
                sl = (buf, r0 * _RU + u, pl.ds(j * _LANES, _LANES))
                rows_ref[sl] = rows_ref[sl] * _SCALE
        return carry
    lax.fori_loop(0, _C // _RU, row_body, 0)


def _body(x_hbm, lut_hbm, out_hbm, idx_t, rows_v, gsem, osem):
    wid = lax.axis_index("s") * 2 + lax.axis_index("c")
    b0 = wid * _BPW

    pltpu.sync_copy(x_hbm.at[pl.ds(0, _SEQ), pl.ds(b0, _BPW)], idx_t)

    def idx_ref(c):
        s = c // _KPS
        k = c % _KPS
        return idx_t.at[s].at[pl.ds(k * _C, _C)]

    def gather_start(c, buf):
        pltpu.async_copy(lut_hbm.at[idx_ref(c)], rows_v.at[buf], gsem)

    def gather_wait(c, buf):
        pltpu.make_async_copy(
            lut_hbm.at[idx_ref(c)], rows_v.at[buf], gsem).wait()

    def out_dst(c):
        s = c // _KPS
        k = c % _KPS
        return out_hbm.at[s].at[pl.ds(b0 + k * _C, _C)]

    def out_start(c, buf):
        pltpu.async_copy(rows_v.at[buf], out_dst(c), osem)

    def out_wait(c, buf):
        pltpu.make_async_copy(rows_v.at[buf], out_dst(c), osem).wait()

    for c in range(_NBUF - 1):
        gather_start(c, c)

    def chunk_step(c, buf):
        gather_wait(c, buf)
        _scale_chunk(rows_v, buf)
        out_start(c, buf)

        nxt = c + _NBUF - 1
        nbuf = (buf + _NBUF - 1) % _NBUF

        @pl.when(c >= 1)
        def _drain_prev_out():
            out_wait(c - 1, nbuf)

        @pl.when(nxt < _NCHUNK)
        def _issue_next():
            gather_start(nxt, nbuf)

    def loop_body(c0, carry):
        for buf in range(_NBUF):
            chunk_step(c0 + buf, buf)
        return carry

    n_main = (_NCHUNK // _NBUF) * _NBUF
    lax.fori_loop(0, _NCHUNK // _NBUF,
                  lambda i, a: loop_body(i * _NBUF, a), 0)
    for c in range(n_main, _NCHUNK):
        chunk_step(c, c % _NBUF)
    out_wait(_NCHUNK - 1, (_NCHUNK - 1) % _NBUF)


@jax.jit
def _lookup(x_flat, lut):
    mesh = plsc.VectorSubcoreMesh(core_axis_name="c", subcore_axis_name="s")
    f = functools.partial(
        pl.kernel,
        mesh=mesh,
        out_type=jax.ShapeDtypeStruct((_SEQ, _BATCH, _D), jnp.float32),
        scratch_types=[
            pltpu.VMEM((_SEQ, _BPW), jnp.int32),
            pltpu.VMEM((_NBUF, _C, _D), jnp.float32),
            pltpu.SemaphoreType.DMA,
            pltpu.SemaphoreType.DMA,
        ],
    )(_body)
    return f(x_flat, lut)


def kernel(x, lut):
    out_t = _lookup(x.T.astype(jnp.int32), lut)
    return out_t.transpose(1, 0, 2)
